# baseline (speedup 1.0000x reference)
import functools

import jax
import jax.numpy as jnp
import numpy as np
from jax import lax
from jax.experimental import pallas as pl
from jax.experimental.pallas import tpu as pltpu

_N_BINS = 15
_N = 100000
_C = 1000
_BLOCK_N = 400
_GRID = _N // _BLOCK_N

_BOUNDS = [float(b) for b in
           np.linspace(0.0, 1.0, _N_BINS + 1).astype(np.float32)[:_N_BINS]]


def _ece_body(probs_ref, labels_ref, out_ref, acc_ref):
    i = pl.program_id(0)

    @pl.when(i == 0)
    def _init():
        acc_ref[...] = jnp.zeros_like(acc_ref)

    x = probs_ref[...]
    conf = jnp.max(x, axis=1, keepdims=True)
    col = lax.broadcasted_iota(jnp.int32, x.shape, 1)
    pred = jnp.min(jnp.where(x == conf, col, _C), axis=1, keepdims=True)
    lbl = labels_ref[0].reshape(_BLOCK_N, 1)
    correct = (pred == lbl).astype(jnp.float32)

    nbelow = jnp.zeros_like(conf, dtype=jnp.int32)
    for b in _BOUNDS:
        nbelow = nbelow + (conf > b).astype(jnp.int32)
    bin_idx = nbelow - 1
    lanes = lax.broadcasted_iota(jnp.int32, (_BLOCK_N, 128), 1)
    onehot = (bin_idx == lanes).astype(jnp.float32)

    acc_ref[0:1, :] += jnp.sum(onehot, axis=0, keepdims=True)
    acc_ref[1:2, :] += jnp.sum(onehot * conf, axis=0, keepdims=True)
    acc_ref[2:3, :] += jnp.sum(onehot * correct, axis=0, keepdims=True)

    @pl.when(i == _GRID - 1)
    def _fin():
        count = acc_ref[0:1, :]
        s_conf = acc_ref[1:2, :]
        s_corr = acc_ref[2:3, :]
        ece = jnp.sum(jnp.abs(s_conf - s_corr), axis=1, keepdims=True)
        out_ref[0:1, :] = jnp.broadcast_to(ece, (1, 128))
        out_ref[1:2, :] = s_corr
        out_ref[2:3, :] = count


@functools.partial(jax.jit, static_argnames=())
def _ece_pallas(probs, labels3):
    out = pl.pallas_call(
        _ece_body,
        grid=(_GRID,),
        in_specs=[
            pl.BlockSpec((_BLOCK_N, _C), lambda i: (i, 0)),
            pl.BlockSpec((1, 1, _BLOCK_N), lambda i: (i, 0, 0)),
        ],
        out_specs=pl.BlockSpec((8, 128), lambda i: (0, 0)),
        out_shape=jax.ShapeDtypeStruct((8, 128), jnp.float32),
        scratch_shapes=[pltpu.VMEM((8, 128), jnp.float32)],
        compiler_params=pltpu.CompilerParams(
            dimension_semantics=("arbitrary",),
        ),
    )(probs, labels3)
    return out


def kernel(probs, labels, mode):
    del mode
    labels3 = labels.reshape(_GRID, 1, _BLOCK_N)
    out = _ece_pallas(probs, labels3)
    ece = out[0, 0:1]
    correct = out[1, 0:_N_BINS]
    num = out[2, 0:_N_BINS]
    return (ece, correct, num)
